# baseline (speedup 1.0000x reference)
import functools

import jax
import jax.numpy as jnp
import numpy as np
from jax.experimental import pallas as pl
from jax.experimental.pallas import tpu as pltpu

K_NEIGH = 20
GROUPS = 8
SCALE = 1.0
NEG_SLOPE = 0.2
BN_SCALE = 1.0 / np.sqrt(1.0 + 1e-5)
NEG_BIG = -3.0e38


def _act(z):
    y = z * BN_SCALE
    return jnp.where(y >= 0, y, NEG_SLOPE * y)


def _attn_layer_body(hd, xn_ref, wqt_ref, wkmt_ref, wvmt_ref, wvdt_ref, o_ref):
    xt = xn_ref[0]
    s = jax.lax.dot_general(xt, xt, (((1,), (1,)), ((), ())),
                            preferred_element_type=jnp.float32)
    xx = jnp.sum(xt * xt, axis=1)
    pw = 2.0 * s - xx[None, :]

    qn = jnp.dot(xt, wqt_ref[...], preferred_element_type=jnp.float32)
    kmn = jnp.dot(xt, wkmt_ref[...], preferred_element_type=jnp.float32)
    vmn = jnp.dot(xt, wvmt_ref[...], preferred_element_type=jnp.float32)
    vdn = jnp.dot(xt, wvdt_ref[...], preferred_element_type=jnp.float32)

    es = []
    for g in range(GROUPS):
        sl = slice(g * hd, (g + 1) * hd)
        es.append(jax.lax.dot_general(
            qn[:, sl], kmn[:, sl], (((1,), (1,)), ((), ())),
            preferred_element_type=jnp.float32))

    m = jnp.max(pw, axis=1, keepdims=True)
    for _ in range(K_NEIGH - 1):
        m = jnp.max(jnp.where(pw < m, pw, NEG_BIG), axis=1, keepdims=True)
    madd = jnp.where(pw >= m, 0.0, NEG_BIG)
    n = xt.shape[0]
    ones_col = jnp.ones((n, 1), dtype=jnp.float32)

    for g in range(GROUPS):
        sl = slice(g * hd, (g + 1) * hd)
        e = es[g]
        qg = qn[:, sl]
        kg = kmn[:, sl]
        kmax2 = jnp.max(jnp.sum(kg * kg, axis=1))
        shift = jnp.sqrt(jnp.sum(qg * qg, axis=1, keepdims=True) * kmax2)
        a = jnp.exp(e - shift + madd)
        vplus = jnp.concatenate([vmn[:, sl], ones_col], axis=1)
        ovz = jnp.dot(a, vplus, preferred_element_type=jnp.float32)
        og = ovz[:, :hd] / ovz[:, hd:hd + 1]
        o_ref[0, :, sl] = _act(og - vdn[:, sl])


def _attn_layer(xn, wq, wk, wv):
    b, n, c = xn.shape
    cout = wq.shape[0]
    hd = cout // GROUPS
    wqt = wq.T * float(SCALE / np.sqrt(hd))
    wkmt = (wk[:, :c] + wk[:, c:]).T
    wvmt = (wv[:, :c] + wv[:, c:]).T
    wvdt = wv[:, :c].T
    body = functools.partial(_attn_layer_body, hd)
    return pl.pallas_call(
        body,
        grid=(b,),
        in_specs=[
            pl.BlockSpec((1, n, c), lambda i: (i, 0, 0)),
            pl.BlockSpec((c, cout), lambda i: (0, 0)),
            pl.BlockSpec((c, cout), lambda i: (0, 0)),
            pl.BlockSpec((c, cout), lambda i: (0, 0)),
            pl.BlockSpec((c, cout), lambda i: (0, 0)),
        ],
        out_specs=pl.BlockSpec((1, n, cout), lambda i: (i, 0, 0)),
        out_shape=jax.ShapeDtypeStruct((b, n, cout), jnp.float32),
        compiler_params=pltpu.CompilerParams(
            dimension_semantics=("parallel",)),
    )(xn, wqt, wkmt, wvmt, wvdt)


def _proj_pool_body(xn_ref, w5t_ref, o_ref):
    xt = xn_ref[0]
    x5 = _act(jnp.dot(xt, w5t_ref[...], preferred_element_type=jnp.float32))
    o_ref[0, 0, :] = jnp.max(x5, axis=0)


def _proj_pool(xn, w5):
    b, n, c = xn.shape
    cout = w5.shape[0]
    return pl.pallas_call(
        _proj_pool_body,
        grid=(b,),
        in_specs=[
            pl.BlockSpec((1, n, c), lambda i: (i, 0, 0)),
            pl.BlockSpec((c, cout), lambda i: (0, 0)),
        ],
        out_specs=pl.BlockSpec((1, 1, cout), lambda i: (i, 0, 0)),
        out_shape=jax.ShapeDtypeStruct((b, 1, cout), jnp.float32),
        compiler_params=pltpu.CompilerParams(
            dimension_semantics=("parallel",)),
    )(xn, w5.T)


def _head_body(p_ref, l1t_ref, l2wt_ref, l2b_ref, l3wt_ref, l3b_ref, o_ref):
    h = _act(jnp.dot(p_ref[...], l1t_ref[...], preferred_element_type=jnp.float32))
    h = _act(jnp.dot(h, l2wt_ref[...], preferred_element_type=jnp.float32)
             + l2b_ref[...])
    o_ref[...] = (jnp.dot(h, l3wt_ref[...], preferred_element_type=jnp.float32)
                  + l3b_ref[...])


def _head(pooled, l1, l2w, l2b, l3w, l3b):
    b = pooled.shape[0]
    return pl.pallas_call(
        _head_body,
        out_shape=jax.ShapeDtypeStruct((b, l3w.shape[0]), jnp.float32),
    )(pooled, l1.T, l2w.T, l2b[None, :], l3w.T, l3b[None, :])


def kernel(x, Wq1, Wk1, Wv1, Wq2, Wk2, Wv2, Wq3, Wk3, Wv3, Wq4, Wk4, Wv4,
           W5, L1, L2w, L2b, L3w, L3b):
    xn = jnp.swapaxes(x, 1, 2)
    x1 = _attn_layer(xn, Wq1, Wk1, Wv1)
    x2 = _attn_layer(x1, Wq2, Wk2, Wv2)
    x3 = _attn_layer(x2, Wq3, Wk3, Wv3)
    x4 = _attn_layer(x3, Wq4, Wk4, Wv4)
    pooled = _proj_pool(x4, W5)[:, 0, :]
    return _head(pooled, L1, L2w, L2b, L3w, L3b)


# speedup vs baseline: 1.2272x; 1.2272x over previous
import functools

import jax
import jax.numpy as jnp
import numpy as np
from jax.experimental import pallas as pl
from jax.experimental.pallas import tpu as pltpu

K_NEIGH = 20
GROUPS = 8
SCALE = 1.0
NEG_SLOPE = 0.2
BN_SCALE = 1.0 / np.sqrt(1.0 + 1e-5)
NEG_BIG = -3.0e38


def _act(z):
    y = z * BN_SCALE
    return jnp.where(y >= 0, y, NEG_SLOPE * y)


def _attn_layer_body(hd, xn_ref, wqt_ref, wkmt_ref, wvmt_ref, wvdt_ref, o_ref):
    xt = xn_ref[0]
    s = jax.lax.dot_general(xt, xt, (((1,), (1,)), ((), ())),
                            preferred_element_type=jnp.float32)
    xx = jnp.sum(xt * xt, axis=1)
    pw = 2.0 * s - xx[None, :]

    qn = jnp.dot(xt, wqt_ref[...], preferred_element_type=jnp.float32)
    kmn = jnp.dot(xt, wkmt_ref[...], preferred_element_type=jnp.float32)
    vmn = jnp.dot(xt, wvmt_ref[...], preferred_element_type=jnp.float32)
    vdn = jnp.dot(xt, wvdt_ref[...], preferred_element_type=jnp.float32)

    es = []
    for g in range(GROUPS):
        sl = slice(g * hd, (g + 1) * hd)
        es.append(jax.lax.dot_general(
            qn[:, sl], kmn[:, sl], (((1,), (1,)), ((), ())),
            preferred_element_type=jnp.float32))

    m = jnp.max(pw, axis=1, keepdims=True)
    for _ in range(K_NEIGH - 1):
        m = jnp.max(jnp.where(pw < m, pw, NEG_BIG), axis=1, keepdims=True)
    madd = jnp.where(pw >= m, 0.0, NEG_BIG)
    n = xt.shape[0]
    ones_col = jnp.ones((n, 1), dtype=jnp.float32)

    for g in range(GROUPS):
        sl = slice(g * hd, (g + 1) * hd)
        e = es[g]
        emax = jnp.max(e + madd, axis=1, keepdims=True)
        a = jnp.exp(e - emax + madd)
        vplus = jnp.concatenate([vmn[:, sl], ones_col], axis=1)
        ovz = jnp.dot(a, vplus, preferred_element_type=jnp.float32)
        og = ovz[:, :hd] / ovz[:, hd:hd + 1]
        o_ref[0, :, sl] = _act(og - vdn[:, sl])


def _attn_pool_body(hd, xn_ref, wqt_ref, wkmt_ref, wvmt_ref, wvdt_ref,
                    w5t_ref, o_ref):
    xt = xn_ref[0]
    s = jax.lax.dot_general(xt, xt, (((1,), (1,)), ((), ())),
                            preferred_element_type=jnp.float32)
    xx = jnp.sum(xt * xt, axis=1)
    pw = 2.0 * s - xx[None, :]

    qn = jnp.dot(xt, wqt_ref[...], preferred_element_type=jnp.float32)
    kmn = jnp.dot(xt, wkmt_ref[...], preferred_element_type=jnp.float32)
    vmn = jnp.dot(xt, wvmt_ref[...], preferred_element_type=jnp.float32)
    vdn = jnp.dot(xt, wvdt_ref[...], preferred_element_type=jnp.float32)

    es = []
    for g in range(GROUPS):
        sl = slice(g * hd, (g + 1) * hd)
        es.append(jax.lax.dot_general(
            qn[:, sl], kmn[:, sl], (((1,), (1,)), ((), ())),
            preferred_element_type=jnp.float32))

    m = jnp.max(pw, axis=1, keepdims=True)
    for _ in range(K_NEIGH - 1):
        m = jnp.max(jnp.where(pw < m, pw, NEG_BIG), axis=1, keepdims=True)
    madd = jnp.where(pw >= m, 0.0, NEG_BIG)
    n = xt.shape[0]
    ones_col = jnp.ones((n, 1), dtype=jnp.float32)

    outs = []
    for g in range(GROUPS):
        sl = slice(g * hd, (g + 1) * hd)
        e = es[g]
        emax = jnp.max(e + madd, axis=1, keepdims=True)
        a = jnp.exp(e - emax + madd)
        vplus = jnp.concatenate([vmn[:, sl], ones_col], axis=1)
        ovz = jnp.dot(a, vplus, preferred_element_type=jnp.float32)
        og = ovz[:, :hd] / ovz[:, hd:hd + 1]
        outs.append(_act(og - vdn[:, sl]))

    x4 = jnp.concatenate(outs, axis=1)
    x5 = _act(jnp.dot(x4, w5t_ref[...], preferred_element_type=jnp.float32))
    o_ref[0, 0, :] = jnp.max(x5, axis=0)


def _attn_pool_layer(xn, wq, wk, wv, w5):
    b, n, c = xn.shape
    cout = wq.shape[0]
    c5 = w5.shape[0]
    hd = cout // GROUPS
    wqt = wq.T * float(SCALE / np.sqrt(hd))
    wkmt = (wk[:, :c] + wk[:, c:]).T
    wvmt = (wv[:, :c] + wv[:, c:]).T
    wvdt = wv[:, :c].T
    body = functools.partial(_attn_pool_body, hd)
    return pl.pallas_call(
        body,
        grid=(b,),
        in_specs=[
            pl.BlockSpec((1, n, c), lambda i: (i, 0, 0)),
            pl.BlockSpec((c, cout), lambda i: (0, 0)),
            pl.BlockSpec((c, cout), lambda i: (0, 0)),
            pl.BlockSpec((c, cout), lambda i: (0, 0)),
            pl.BlockSpec((c, cout), lambda i: (0, 0)),
            pl.BlockSpec((cout, c5), lambda i: (0, 0)),
        ],
        out_specs=pl.BlockSpec((1, 1, c5), lambda i: (i, 0, 0)),
        out_shape=jax.ShapeDtypeStruct((b, 1, c5), jnp.float32),
        compiler_params=pltpu.CompilerParams(
            dimension_semantics=("parallel",)),
    )(xn, wqt, wkmt, wvmt, wvdt, w5.T)


def _attn_layer(xn, wq, wk, wv):
    b, n, c = xn.shape
    cout = wq.shape[0]
    hd = cout // GROUPS
    wqt = wq.T * float(SCALE / np.sqrt(hd))
    wkmt = (wk[:, :c] + wk[:, c:]).T
    wvmt = (wv[:, :c] + wv[:, c:]).T
    wvdt = wv[:, :c].T
    body = functools.partial(_attn_layer_body, hd)
    return pl.pallas_call(
        body,
        grid=(b,),
        in_specs=[
            pl.BlockSpec((1, n, c), lambda i: (i, 0, 0)),
            pl.BlockSpec((c, cout), lambda i: (0, 0)),
            pl.BlockSpec((c, cout), lambda i: (0, 0)),
            pl.BlockSpec((c, cout), lambda i: (0, 0)),
            pl.BlockSpec((c, cout), lambda i: (0, 0)),
        ],
        out_specs=pl.BlockSpec((1, n, cout), lambda i: (i, 0, 0)),
        out_shape=jax.ShapeDtypeStruct((b, n, cout), jnp.float32),
        compiler_params=pltpu.CompilerParams(
            dimension_semantics=("parallel",)),
    )(xn, wqt, wkmt, wvmt, wvdt)


def _proj_pool_body(xn_ref, w5t_ref, o_ref):
    xt = xn_ref[0]
    x5 = _act(jnp.dot(xt, w5t_ref[...], preferred_element_type=jnp.float32))
    o_ref[0, 0, :] = jnp.max(x5, axis=0)


def _proj_pool(xn, w5):
    b, n, c = xn.shape
    cout = w5.shape[0]
    return pl.pallas_call(
        _proj_pool_body,
        grid=(b,),
        in_specs=[
            pl.BlockSpec((1, n, c), lambda i: (i, 0, 0)),
            pl.BlockSpec((c, cout), lambda i: (0, 0)),
        ],
        out_specs=pl.BlockSpec((1, 1, cout), lambda i: (i, 0, 0)),
        out_shape=jax.ShapeDtypeStruct((b, 1, cout), jnp.float32),
        compiler_params=pltpu.CompilerParams(
            dimension_semantics=("parallel",)),
    )(xn, w5.T)


def _head_body(p_ref, l1t_ref, l2wt_ref, l2b_ref, l3wt_ref, l3b_ref, o_ref):
    h = _act(jnp.dot(p_ref[...], l1t_ref[...], preferred_element_type=jnp.float32))
    h = _act(jnp.dot(h, l2wt_ref[...], preferred_element_type=jnp.float32)
             + l2b_ref[...])
    o_ref[...] = (jnp.dot(h, l3wt_ref[...], preferred_element_type=jnp.float32)
                  + l3b_ref[...])


def _head(pooled, l1, l2w, l2b, l3w, l3b):
    b = pooled.shape[0]
    return pl.pallas_call(
        _head_body,
        out_shape=jax.ShapeDtypeStruct((b, l3w.shape[0]), jnp.float32),
    )(pooled, l1.T, l2w.T, l2b[None, :], l3w.T, l3b[None, :])


def kernel(x, Wq1, Wk1, Wv1, Wq2, Wk2, Wv2, Wq3, Wk3, Wv3, Wq4, Wk4, Wv4,
           W5, L1, L2w, L2b, L3w, L3b):
    xn = jnp.swapaxes(x, 1, 2)
    x1 = _attn_layer(xn, Wq1, Wk1, Wv1)
    x2 = _attn_layer(x1, Wq2, Wk2, Wv2)
    x3 = _attn_layer(x2, Wq3, Wk3, Wv3)
    pooled = _attn_pool_layer(x3, Wq4, Wk4, Wv4, W5)[:, 0, :]
    return _head(pooled, L1, L2w, L2b, L3w, L3b)
